# speedup vs baseline: 1.7064x; 1.7064x over previous
import jax
import jax.numpy as jnp
from jax import lax
from jax.experimental import pallas as pl
from jax.experimental.pallas import tpu as pltpu
from jax.experimental.pallas import tpu_sc as plsc

NUM_CORES = 2
NUM_SUBCORES = 16
LANES = 16
NUM_WORKERS = NUM_CORES * NUM_SUBCORES

BSZ = 4
SEQ_LEN = 4096
EMBED_DIM = 1024
TOTAL_ROWS = BSZ * SEQ_LEN
ROWS_PER_WORKER = TOTAL_ROWS // NUM_WORKERS
CHUNK = 32
NUM_CHUNKS = ROWS_PER_WORKER // CHUNK


def _sc_body(inp_hbm, w_hbm, out_hbm, inp_v, idx_v, rows_v, sem):
    wid = lax.axis_index("s") * NUM_CORES + lax.axis_index("c")
    base = wid * ROWS_PER_WORKER
    base_s = lax.rem(base, SEQ_LEN)

    pltpu.sync_copy(inp_hbm.at[pl.ds(base, ROWS_PER_WORKER)], inp_v)

    lane = lax.iota(jnp.int32, LANES)
    for v in range(ROWS_PER_WORKER // LANES):
        tok = inp_v[pl.ds(v * LANES, LANES)]
        pos = lane + (base_s + v * LANES + 1)
        idx_v[pl.ds(v * LANES, LANES)] = jnp.where(tok != 0, pos, 0)

    for g in range(NUM_CHUNKS):
        pltpu.async_copy(
            w_hbm.at[idx_v.at[pl.ds(g * CHUNK, CHUNK)]], rows_v, sem
        ).wait()
        pltpu.sync_copy(rows_v, out_hbm.at[pl.ds(base + g * CHUNK, CHUNK)])


@jax.jit
def _sc_embed(inp_flat, weights):
    mesh = plsc.VectorSubcoreMesh(core_axis_name="c", subcore_axis_name="s")
    k = pl.kernel(
        _sc_body,
        out_type=jax.ShapeDtypeStruct((TOTAL_ROWS, EMBED_DIM), jnp.float32),
        mesh=mesh,
        scratch_types=[
            pltpu.VMEM((ROWS_PER_WORKER,), jnp.int32),
            pltpu.VMEM((ROWS_PER_WORKER,), jnp.int32),
            pltpu.VMEM((CHUNK, EMBED_DIM), jnp.float32),
            pltpu.SemaphoreType.DMA,
        ],
    )
    return k(inp_flat, weights)


def kernel(input, weights):
    inp_flat = input.reshape(-1)
    out = _sc_embed(inp_flat, weights)
    return out.reshape(BSZ, SEQ_LEN, EMBED_DIM)


# speedup vs baseline: 1.9213x; 1.1259x over previous
import jax
import jax.numpy as jnp
from jax import lax
from jax.experimental import pallas as pl
from jax.experimental.pallas import tpu as pltpu
from jax.experimental.pallas import tpu_sc as plsc

NUM_CORES = 2
NUM_SUBCORES = 16
LANES = 16
NUM_WORKERS = NUM_CORES * NUM_SUBCORES

BSZ = 4
SEQ_LEN = 4096
EMBED_DIM = 1024
TOTAL_ROWS = BSZ * SEQ_LEN
ROWS_PER_WORKER = TOTAL_ROWS // NUM_WORKERS
CHUNK = 32
NUM_CHUNKS = ROWS_PER_WORKER // CHUNK


def _sc_body(inp_hbm, w_hbm, out_hbm, inp_v, idx_v, rows0, rows1,
             gsem0, gsem1, ssem0, ssem1):
    wid = lax.axis_index("s") * NUM_CORES + lax.axis_index("c")
    base = wid * ROWS_PER_WORKER
    base_s = lax.rem(base, SEQ_LEN)

    pltpu.sync_copy(inp_hbm.at[pl.ds(base, ROWS_PER_WORKER)], inp_v)

    lane = lax.iota(jnp.int32, LANES)
    for v in range(ROWS_PER_WORKER // LANES):
        tok = inp_v[pl.ds(v * LANES, LANES)]
        pos = lane + (base_s + v * LANES + 1)
        idx_v[pl.ds(v * LANES, LANES)] = jnp.where(tok != 0, pos, 0)

    bufs = (rows0, rows1)
    gsems = (gsem0, gsem1)
    ssems = (ssem0, ssem1)

    def gather(g):
        return pltpu.async_copy(
            w_hbm.at[idx_v.at[pl.ds(g * CHUNK, CHUNK)]],
            bufs[g % 2], gsems[g % 2])

    def scatter(g):
        return pltpu.async_copy(
            bufs[g % 2], out_hbm.at[pl.ds(base + g * CHUNK, CHUNK)],
            ssems[g % 2])

    pending_scatter = [None, None]
    pending_gather = gather(0)
    for g in range(NUM_CHUNKS):
        b = g % 2
        pending_gather.wait()
        if g + 1 < NUM_CHUNKS:
            nb = (g + 1) % 2
            if pending_scatter[nb] is not None:
                pending_scatter[nb].wait()
            pending_gather = gather(g + 1)
        pending_scatter[b] = scatter(g)
    for s in pending_scatter:
        if s is not None:
            s.wait()


@jax.jit
def _sc_embed(inp_flat, weights):
    mesh = plsc.VectorSubcoreMesh(core_axis_name="c", subcore_axis_name="s")
    k = pl.kernel(
        _sc_body,
        out_type=jax.ShapeDtypeStruct((TOTAL_ROWS, EMBED_DIM), jnp.float32),
        mesh=mesh,
        scratch_types=[
            pltpu.VMEM((ROWS_PER_WORKER,), jnp.int32),
            pltpu.VMEM((ROWS_PER_WORKER,), jnp.int32),
            pltpu.VMEM((CHUNK, EMBED_DIM), jnp.float32),
            pltpu.VMEM((CHUNK, EMBED_DIM), jnp.float32),
            pltpu.SemaphoreType.DMA,
            pltpu.SemaphoreType.DMA,
            pltpu.SemaphoreType.DMA,
            pltpu.SemaphoreType.DMA,
        ],
    )
    return k(inp_flat, weights)


def kernel(input, weights):
    inp_flat = input.reshape(-1)
    out = _sc_embed(inp_flat, weights)
    return out.reshape(BSZ, SEQ_LEN, EMBED_DIM)


# speedup vs baseline: 2.6999x; 1.4052x over previous
import jax
import jax.numpy as jnp
from jax import lax
from jax.experimental import pallas as pl
from jax.experimental.pallas import tpu as pltpu
from jax.experimental.pallas import tpu_sc as plsc

NUM_CORES = 2
NUM_SUBCORES = 16
LANES = 16
NUM_WORKERS = NUM_CORES * NUM_SUBCORES

BSZ = 4
SEQ_LEN = 4096
EMBED_DIM = 1024
TOTAL_ROWS = BSZ * SEQ_LEN
POS_PER_WORKER = SEQ_LEN // NUM_WORKERS
POS_CHUNK = 32
NUM_CHUNKS = POS_PER_WORKER // POS_CHUNK
VREGS_PER_SLICE = POS_PER_WORKER // LANES


def _lane_shuffle(x, idx):
    return lax.gather(
        x, idx[:, None],
        lax.GatherDimensionNumbers(
            offset_dims=(), collapsed_slice_dims=(0,), start_index_map=(0,)),
        (1,), mode=lax.GatherScatterMode.PROMISE_IN_BOUNDS)


def _sc_body(inp_hbm, w_hbm, out_hbm, inp_v, idx_v, fix_idx, fixbuf,
             rows0, rows1, gsem, ssem0, ssem1, fsem):
    wid = lax.axis_index("s") * NUM_CORES + lax.axis_index("c")
    pos0 = wid * POS_PER_WORKER

    bufs = (rows0, rows1)
    ssems = (ssem0, ssem1)
    lane = lax.iota(jnp.int32, LANES)

    for b in range(BSZ):
        pltpu.sync_copy(
            inp_hbm.at[pl.ds(b * SEQ_LEN + pos0, POS_PER_WORKER)],
            inp_v.at[pl.ds(b * POS_PER_WORKER, POS_PER_WORKER)])

    def drain_scatters(c):
        for _ in range(BSZ):
            pltpu.make_async_copy(
                out_hbm.at[pl.ds(0, POS_CHUNK)], bufs[c % 2],
                ssems[c % 2]).wait()

    for c in range(NUM_CHUNKS):
        buf = bufs[c % 2]
        ssem = ssems[c % 2]
        if c >= 2:
            drain_scatters(c - 2)
        for v in range(POS_CHUNK // LANES):
            idx_v[pl.ds(v * LANES, LANES)] = (
                lane + (pos0 + c * POS_CHUNK + v * LANES + 1))
        pltpu.async_copy(w_hbm.at[idx_v], buf, gsem).wait()
        for b in range(BSZ):
            pltpu.async_copy(
                buf,
                out_hbm.at[pl.ds(b * SEQ_LEN + pos0 + c * POS_CHUNK,
                                 POS_CHUNK)],
                ssem)
    drain_scatters(NUM_CHUNKS - 2)
    drain_scatters(NUM_CHUNKS - 1)

    def any_true(mask_i32):
        m = mask_i32
        for sh in (1, 2, 4, 8):
            m = m | _lane_shuffle(m, lane ^ sh)
        return m[0] > 0

    toks = [
        inp_v[pl.ds(b * POS_PER_WORKER + v * LANES, LANES)]
        for b in range(BSZ) for v in range(VREGS_PER_SLICE)
    ]
    worker_acc = toks[0] == 0
    for t in toks[1:]:
        worker_acc = worker_acc | (t == 0)

    @pl.when(any_true(jnp.where(worker_acc, 1, 0)))
    def _fix_worker():
        for b in range(BSZ):
            for v in range(VREGS_PER_SLICE):
                tok = inp_v[pl.ds(b * POS_PER_WORKER + v * LANES, LANES)]

                @pl.when(any_true(jnp.where(tok == 0, 1, 0)))
                def _fix(tok=tok, b=b, v=v):
                    fix_idx[...] = jnp.where(
                        tok == 0, 0, lane + (pos0 + v * LANES + 1))
                    pltpu.async_copy(w_hbm.at[fix_idx], fixbuf, fsem).wait()
                    pltpu.sync_copy(
                        fixbuf,
                        out_hbm.at[pl.ds(
                            b * SEQ_LEN + pos0 + v * LANES, LANES)])


@jax.jit
def _sc_embed(inp_flat, weights):
    mesh = plsc.VectorSubcoreMesh(core_axis_name="c", subcore_axis_name="s")
    k = pl.kernel(
        _sc_body,
        out_type=jax.ShapeDtypeStruct((TOTAL_ROWS, EMBED_DIM), jnp.float32),
        mesh=mesh,
        scratch_types=[
            pltpu.VMEM((BSZ * POS_PER_WORKER,), jnp.int32),
            pltpu.VMEM((POS_CHUNK,), jnp.int32),
            pltpu.VMEM((LANES,), jnp.int32),
            pltpu.VMEM((LANES, EMBED_DIM), jnp.float32),
            pltpu.VMEM((POS_CHUNK, EMBED_DIM), jnp.float32),
            pltpu.VMEM((POS_CHUNK, EMBED_DIM), jnp.float32),
            pltpu.SemaphoreType.DMA,
            pltpu.SemaphoreType.DMA,
            pltpu.SemaphoreType.DMA,
            pltpu.SemaphoreType.DMA,
        ],
    )
    return k(inp_flat, weights)


def kernel(input, weights):
    inp_flat = input.reshape(-1)
    out = _sc_embed(inp_flat, weights)
    return out.reshape(BSZ, SEQ_LEN, EMBED_DIM)


# speedup vs baseline: 2.7891x; 1.0331x over previous
import jax
import jax.numpy as jnp
from jax import lax
from jax.experimental import pallas as pl
from jax.experimental.pallas import tpu as pltpu
from jax.experimental.pallas import tpu_sc as plsc

NUM_CORES = 2
NUM_SUBCORES = 16
LANES = 16
NUM_WORKERS = NUM_CORES * NUM_SUBCORES

BSZ = 4
SEQ_LEN = 4096
EMBED_DIM = 1024
TOTAL_ROWS = BSZ * SEQ_LEN
POS_PER_WORKER = SEQ_LEN // NUM_WORKERS
POS_CHUNK = 32
NUM_CHUNKS = POS_PER_WORKER // POS_CHUNK
VREGS_PER_SLICE = POS_PER_WORKER // LANES


def _lane_shuffle(x, idx):
    return lax.gather(
        x, idx[:, None],
        lax.GatherDimensionNumbers(
            offset_dims=(), collapsed_slice_dims=(0,), start_index_map=(0,)),
        (1,), mode=lax.GatherScatterMode.PROMISE_IN_BOUNDS)


def _sc_body(inp_hbm, w_hbm, out_hbm, inp_v, idx_v, fix_idx, fixbuf,
             rows0, rows1, gsem, ssem0, ssem1, fsem):
    wid = lax.axis_index("s") * NUM_CORES + lax.axis_index("c")
    pos0 = wid * POS_PER_WORKER

    bufs = (rows0, rows1)
    ssems = (ssem0, ssem1)
    lane = lax.iota(jnp.int32, LANES)
    HALF = POS_CHUNK // 2

    for v in range(POS_PER_WORKER // LANES):
        idx_v[pl.ds(v * LANES, LANES)] = lane + (pos0 + v * LANES + 1)

    tok_copies = [
        pltpu.async_copy(
            inp_hbm.at[pl.ds(b * SEQ_LEN + pos0, POS_PER_WORKER)],
            inp_v.at[pl.ds(b * POS_PER_WORKER, POS_PER_WORKER)], fsem)
        for b in range(BSZ)
    ]

    def drain_scatters(c):
        for _ in range(2 * BSZ):
            pltpu.make_async_copy(
                out_hbm.at[pl.ds(0, HALF)],
                bufs[c % 2].at[pl.ds(0, HALF)],
                ssems[c % 2]).wait()

    for c in range(NUM_CHUNKS):
        buf = bufs[c % 2]
        ssem = ssems[c % 2]
        if c >= 2:
            drain_scatters(c - 2)
        ghs = [
            pltpu.async_copy(
                w_hbm.at[idx_v.at[pl.ds(c * POS_CHUNK + h * HALF, HALF)]],
                buf.at[pl.ds(h * HALF, HALF)], gsem)
            for h in range(2)
        ]
        for h in range(2):
            ghs[h].wait()
            for b in range(BSZ):
                pltpu.async_copy(
                    buf.at[pl.ds(h * HALF, HALF)],
                    out_hbm.at[pl.ds(
                        b * SEQ_LEN + pos0 + c * POS_CHUNK + h * HALF,
                        HALF)],
                    ssem)
    drain_scatters(NUM_CHUNKS - 2)
    drain_scatters(NUM_CHUNKS - 1)
    for h in tok_copies:
        h.wait()

    def any_true(mask_i32):
        m = mask_i32
        for sh in (1, 2, 4, 8):
            m = m | _lane_shuffle(m, lane ^ sh)
        return m[0] > 0

    toks = [
        inp_v[pl.ds(b * POS_PER_WORKER + v * LANES, LANES)]
        for b in range(BSZ) for v in range(VREGS_PER_SLICE)
    ]
    worker_acc = toks[0] == 0
    for t in toks[1:]:
        worker_acc = worker_acc | (t == 0)

    @pl.when(any_true(jnp.where(worker_acc, 1, 0)))
    def _fix_worker():
        for b in range(BSZ):
            for v in range(VREGS_PER_SLICE):
                tok = inp_v[pl.ds(b * POS_PER_WORKER + v * LANES, LANES)]

                @pl.when(any_true(jnp.where(tok == 0, 1, 0)))
                def _fix(tok=tok, b=b, v=v):
                    fix_idx[...] = jnp.where(
                        tok == 0, 0, lane + (pos0 + v * LANES + 1))
                    pltpu.async_copy(w_hbm.at[fix_idx], fixbuf, fsem).wait()
                    pltpu.sync_copy(
                        fixbuf,
                        out_hbm.at[pl.ds(
                            b * SEQ_LEN + pos0 + v * LANES, LANES)])


@jax.jit
def _sc_embed(inp_flat, weights):
    mesh = plsc.VectorSubcoreMesh(core_axis_name="c", subcore_axis_name="s")
    k = pl.kernel(
        _sc_body,
        out_type=jax.ShapeDtypeStruct((TOTAL_ROWS, EMBED_DIM), jnp.float32),
        mesh=mesh,
        scratch_types=[
            pltpu.VMEM((BSZ * POS_PER_WORKER,), jnp.int32),
            pltpu.VMEM((POS_PER_WORKER,), jnp.int32),
            pltpu.VMEM((LANES,), jnp.int32),
            pltpu.VMEM((LANES, EMBED_DIM), jnp.float32),
            pltpu.VMEM((POS_CHUNK, EMBED_DIM), jnp.float32),
            pltpu.VMEM((POS_CHUNK, EMBED_DIM), jnp.float32),
            pltpu.SemaphoreType.DMA,
            pltpu.SemaphoreType.DMA,
            pltpu.SemaphoreType.DMA,
            pltpu.SemaphoreType.DMA,
        ],
    )
    return k(inp_flat, weights)


def kernel(input, weights):
    inp_flat = input.reshape(-1)
    out = _sc_embed(inp_flat, weights)
    return out.reshape(BSZ, SEQ_LEN, EMBED_DIM)
